# Initial kernel scaffold; baseline (speedup 1.0000x reference)
import jax
import jax.numpy as jnp
from jax.experimental import pallas as pl


def kernel(x, w1p, b1p, w2p, b2p, wfc_p, bfc_p):
    raise NotImplementedError("write your pallas kernel here")


# speedup vs baseline: 14.7182x; 14.7182x over previous
import jax
import jax.numpy as jnp
from jax.experimental import pallas as pl
from jax.experimental.pallas import tpu as pltpu

_LANES = 128


def _cnn_kernel(x_ref, w1_ref, b1_ref, w2_ref, b2_ref, off2_ref, wfc_ref,
                bfc_ref, out_ref, xpad_ref, h1_ref, t1_ref, t2_ref, f_ref):

    xpad_ref[...] = jnp.zeros_like(xpad_ref)
    h1_ref[...] = jnp.zeros_like(h1_ref)

    for i in range(28):
        xpad_ref[pl.ds(30 * (i + 1) + 1, 28), :] = x_ref[pl.ds(28 * i, 28), :]

    for a in range(14):
        base = 60 * a + 31
        accs = [None] * 8
        for t in range(9):
            dy, dx = divmod(t, 3)
            xs = xpad_ref[pl.ds(base + 30 * (dy - 1) + (dx - 1), 60), :]
            for c in range(8):
                p = w1_ref[t, c] * xs
                accs[c] = p if accs[c] is None else accs[c] + p
        for c in range(8):
            t1_ref[c, pl.ds(0, 60), :] = accs[c]
        for c in range(8):
            m = jnp.maximum(
                jnp.maximum(t1_ref[c, pl.ds(0, 14, stride=2), :],
                            t1_ref[c, pl.ds(1, 14, stride=2), :]),
                jnp.maximum(t1_ref[c, pl.ds(30, 14, stride=2), :],
                            t1_ref[c, pl.ds(31, 14, stride=2), :]))
            h1_ref[pl.ds(c * 264 + 16 * (a + 1) + 1, 14), :] = (
                jnp.maximum(m + b1_ref[0, c], 0.0))

    for a in range(7):
        def body(k, accs):
            xs = h1_ref[pl.ds(off2_ref[0, k] + 32 * a, 32), :]
            return tuple(accs[co] + w2_ref[k, co] * xs for co in range(16))

        zero = jnp.zeros((32, _LANES), jnp.float32)
        accs = jax.lax.fori_loop(0, 72, body, (zero,) * 16)
        for co in range(16):
            t2_ref[co, pl.ds(0, 32), :] = accs[co]
        for co in range(16):
            m = jnp.maximum(
                jnp.maximum(t2_ref[co, pl.ds(0, 7, stride=2), :],
                            t2_ref[co, pl.ds(1, 7, stride=2), :]),
                jnp.maximum(t2_ref[co, pl.ds(16, 7, stride=2), :],
                            t2_ref[co, pl.ds(17, 7, stride=2), :]))
            f_ref[pl.ds(co * 49 + 7 * a, 7), :] = (
                jnp.maximum(m + b2_ref[0, co], 0.0))

    out_ref[...] = jnp.dot(wfc_ref[...], f_ref[...],
                           preferred_element_type=jnp.float32) + bfc_ref[...]


def _forward(x, w1p, b1p, w2p, b2p, wfc_p, bfc_p):
    B = x.shape[0]
    G = B // _LANES

    xT = jnp.transpose(x.reshape(B, 784))
    w1s = w1p[:, 0, :8]
    b1s = b1p[:, :8]
    w2s = jnp.transpose(w2p[:, :, :16], (1, 0, 2)).reshape(72, 16)
    b2s = b2p[:, :16]
    ci = jnp.arange(72, dtype=jnp.int32) // 9
    t = jnp.arange(72, dtype=jnp.int32) % 9
    off2 = (ci * 264 + 17 + 16 * (t // 3 - 1) + (t % 3 - 1)).reshape(1, 72)
    wfc_t = jnp.transpose(wfc_p.reshape(49, 16, 10), (2, 1, 0)).reshape(10, 784)
    wfc16 = jnp.pad(wfc_t, ((0, 6), (0, 0)))
    bfc16 = jnp.pad(bfc_p, ((0, 0), (0, 6)))
    bfcv = jnp.broadcast_to(bfc16.reshape(16, 1), (16, _LANES))

    out = pl.pallas_call(
        _cnn_kernel,
        out_shape=jax.ShapeDtypeStruct((G, 16, _LANES), jnp.float32),
        grid=(G,),
        in_specs=[
            pl.BlockSpec((784, _LANES), lambda g: (0, g)),
            pl.BlockSpec(memory_space=pltpu.SMEM),
            pl.BlockSpec(memory_space=pltpu.SMEM),
            pl.BlockSpec(memory_space=pltpu.SMEM),
            pl.BlockSpec(memory_space=pltpu.SMEM),
            pl.BlockSpec(memory_space=pltpu.SMEM),
            pl.BlockSpec((16, 784), lambda g: (0, 0)),
            pl.BlockSpec((16, _LANES), lambda g: (0, 0)),
        ],
        out_specs=pl.BlockSpec((None, 16, _LANES), lambda g: (g, 0, 0)),
        scratch_shapes=[
            pltpu.VMEM((904, _LANES), jnp.float32),
            pltpu.VMEM((2112, _LANES), jnp.float32),
            pltpu.VMEM((8, 64, _LANES), jnp.float32),
            pltpu.VMEM((16, 32, _LANES), jnp.float32),
            pltpu.VMEM((784, _LANES), jnp.float32),
        ],
        compiler_params=pltpu.CompilerParams(
            dimension_semantics=("arbitrary",)),
    )(xT, w1s, b1s, w2s, b2s, off2, wfc16, bfcv)

    return jnp.transpose(out, (0, 2, 1)).reshape(B, 16)[:, :10]


_forward_jit = jax.jit(_forward)


def kernel(x, w1p, b1p, w2p, b2p, wfc_p, bfc_p):
    return _forward_jit(x, w1p, b1p, w2p, b2p, wfc_p, bfc_p)


# speedup vs baseline: 22.4555x; 1.5257x over previous
import jax
import jax.numpy as jnp
from jax.experimental import pallas as pl
from jax.experimental.pallas import tpu as pltpu

_LANES = 128


def _cnn_kernel(x_ref, w1_ref, b1_ref, w2_ref, b2_ref, off2_ref, wfc_ref,
                bfc_ref, out_ref, xpad_ref, h1_ref, t1_ref, t2_ref, f_ref):

    xpad_ref[...] = jnp.zeros_like(xpad_ref)
    h1_ref[...] = jnp.zeros_like(h1_ref)

    for i in range(28):
        xpad_ref[pl.ds(30 * (i + 1) + 1, 28), :] = x_ref[pl.ds(28 * i, 28), :]

    for a in range(14):
        base = 60 * a + 31
        accs = [None] * 8
        for t in range(9):
            dy, dx = divmod(t, 3)
            xs = xpad_ref[pl.ds(base + 30 * (dy - 1) + (dx - 1), 60), :]
            for c in range(8):
                p = w1_ref[t, c] * xs
                accs[c] = p if accs[c] is None else accs[c] + p
        for c in range(8):
            t1_ref[c, pl.ds(0, 60), :] = accs[c]
        for c in range(8):
            m = jnp.maximum(
                jnp.maximum(t1_ref[c, pl.ds(0, 14, stride=2), :],
                            t1_ref[c, pl.ds(1, 14, stride=2), :]),
                jnp.maximum(t1_ref[c, pl.ds(30, 14, stride=2), :],
                            t1_ref[c, pl.ds(31, 14, stride=2), :]))
            h1_ref[pl.ds(c * 264 + 16 * (a + 1) + 1, 14), :] = (
                jnp.maximum(m + b1_ref[0, c], 0.0))

    for a in range(7):
        def body(t, accs):
            accs = list(accs)
            for ci in range(8):
                xs = h1_ref[pl.ds(off2_ref[0, t] + ci * 264 + 32 * a, 32), :]
                for co in range(16):
                    accs[co] = accs[co] + w2_ref[t, ci * 16 + co] * xs
            return tuple(accs)

        zero = jnp.zeros((32, _LANES), jnp.float32)
        accs = jax.lax.fori_loop(0, 9, body, (zero,) * 16)
        for co in range(16):
            t2_ref[co, pl.ds(0, 32), :] = accs[co]
        for co in range(16):
            m = jnp.maximum(
                jnp.maximum(t2_ref[co, pl.ds(0, 7, stride=2), :],
                            t2_ref[co, pl.ds(1, 7, stride=2), :]),
                jnp.maximum(t2_ref[co, pl.ds(16, 7, stride=2), :],
                            t2_ref[co, pl.ds(17, 7, stride=2), :]))
            f_ref[pl.ds(co * 49 + 7 * a, 7), :] = (
                jnp.maximum(m + b2_ref[0, co], 0.0))

    out_ref[...] = jnp.dot(wfc_ref[...], f_ref[...],
                           preferred_element_type=jnp.float32) + bfc_ref[...]


def _forward(x, w1p, b1p, w2p, b2p, wfc_p, bfc_p):
    B = x.shape[0]
    G = B // _LANES

    xT = jnp.transpose(x.reshape(B, 784))
    w1s = w1p[:, 0, :8]
    b1s = b1p[:, :8]
    w2s = w2p[:, :, :16].reshape(9, 128)
    b2s = b2p[:, :16]
    t = jnp.arange(9, dtype=jnp.int32)
    off2 = (17 + 16 * (t // 3 - 1) + (t % 3 - 1)).reshape(1, 9)
    wfc_t = jnp.transpose(wfc_p.reshape(49, 16, 10), (2, 1, 0)).reshape(10, 784)
    wfc16 = jnp.pad(wfc_t, ((0, 6), (0, 0)))
    bfc16 = jnp.pad(bfc_p, ((0, 0), (0, 6)))
    bfcv = jnp.broadcast_to(bfc16.reshape(16, 1), (16, _LANES))

    out = pl.pallas_call(
        _cnn_kernel,
        out_shape=jax.ShapeDtypeStruct((G, 16, _LANES), jnp.float32),
        grid=(G,),
        in_specs=[
            pl.BlockSpec((784, _LANES), lambda g: (0, g)),
            pl.BlockSpec(memory_space=pltpu.SMEM),
            pl.BlockSpec(memory_space=pltpu.SMEM),
            pl.BlockSpec(memory_space=pltpu.SMEM),
            pl.BlockSpec(memory_space=pltpu.SMEM),
            pl.BlockSpec(memory_space=pltpu.SMEM),
            pl.BlockSpec((16, 784), lambda g: (0, 0)),
            pl.BlockSpec((16, _LANES), lambda g: (0, 0)),
        ],
        out_specs=pl.BlockSpec((None, 16, _LANES), lambda g: (g, 0, 0)),
        scratch_shapes=[
            pltpu.VMEM((904, _LANES), jnp.float32),
            pltpu.VMEM((2112, _LANES), jnp.float32),
            pltpu.VMEM((8, 64, _LANES), jnp.float32),
            pltpu.VMEM((16, 32, _LANES), jnp.float32),
            pltpu.VMEM((784, _LANES), jnp.float32),
        ],
        compiler_params=pltpu.CompilerParams(
            dimension_semantics=("arbitrary",)),
    )(xT, w1s, b1s, w2s, b2s, off2, wfc16, bfcv)

    return jnp.transpose(out, (0, 2, 1)).reshape(B, 16)[:, :10]


_forward_jit = jax.jit(_forward)


def kernel(x, w1p, b1p, w2p, b2p, wfc_p, bfc_p):
    return _forward_jit(x, w1p, b1p, w2p, b2p, wfc_p, bfc_p)


# speedup vs baseline: 68.3741x; 3.0449x over previous
import jax
import jax.numpy as jnp
from jax.experimental import pallas as pl
from jax.experimental.pallas import tpu as pltpu

_N = 256


def _cnn_kernel(x_ref, w1b_ref, b1_ref, w2b_ref, b2_ref, wfc_ref, bfc_ref,
                out_ref, xpad_ref, h1_ref, s1_ref, o1_ref, s2_ref, o2_ref,
                f_ref):

    xpad_ref[...] = jnp.zeros_like(xpad_ref)
    h1_ref[...] = jnp.zeros_like(h1_ref)
    s1_ref[:, pl.ds(540, 4), :] = jnp.zeros((2, 4, _N), jnp.bfloat16)

    for i in range(28):
        xpad_ref[pl.ds(30 * (i + 1) + 1, 28), :] = x_ref[pl.ds(28 * i, 28), :]

    for a in range(14):
        u = a % 2
        base = 60 * a + 31
        for t in range(9):
            dy, dx = divmod(t, 3)
            s1_ref[u, pl.ds(60 * t, 60), :] = (
                xpad_ref[pl.ds(base + 30 * (dy - 1) + (dx - 1), 60), :]
                .astype(jnp.bfloat16))
        o1_ref[u, ...] = jnp.dot(w1b_ref[...], s1_ref[u, ...],
                                 preferred_element_type=jnp.float32)
        m = jnp.maximum(
            jnp.maximum(o1_ref[u, pl.ds(0, 112), :],
                        o1_ref[u, pl.ds(112, 112), :]),
            jnp.maximum(o1_ref[u, pl.ds(224, 112), :],
                        o1_ref[u, pl.ds(336, 112), :]))
        for c in range(8):
            h1_ref[pl.ds(c * 264 + 16 * (a + 1) + 1, 14), :] = (
                jnp.maximum(m[c * 14:c * 14 + 14] + b1_ref[0, c], 0.0))

    for a in range(7):
        u = a % 2
        for ci in range(8):
            for t in range(9):
                dy, dx = divmod(t, 3)
                src = ci * 264 + 17 + 16 * (dy - 1) + (dx - 1) + 32 * a
                s2_ref[u, pl.ds(32 * (ci * 9 + t), 32), :] = (
                    h1_ref[pl.ds(src, 32), :].astype(jnp.bfloat16))
        o2_ref[u, ...] = jnp.dot(w2b_ref[...], s2_ref[u, ...],
                                 preferred_element_type=jnp.float32)
        m = jnp.maximum(
            jnp.maximum(o2_ref[u, pl.ds(0, 112), :],
                        o2_ref[u, pl.ds(112, 112), :]),
            jnp.maximum(o2_ref[u, pl.ds(224, 112), :],
                        o2_ref[u, pl.ds(336, 112), :]))
        for co in range(16):
            f_ref[pl.ds(co * 49 + 7 * a, 7), :] = (
                jnp.maximum(m[co * 7:co * 7 + 7] + b2_ref[0, co],
                            0.0).astype(jnp.bfloat16))

    out_ref[...] = jnp.dot(wfc_ref[...], f_ref[...],
                           preferred_element_type=jnp.float32) + bfc_ref[...]


def _forward(x, w1p, b1p, w2p, b2p, wfc_p, bfc_p):
    B = x.shape[0]
    G = B // _N
    bf16 = jnp.bfloat16

    xT = jnp.transpose(x.reshape(B, 784))
    b1s = b1p[:, :8]
    b2s = b2p[:, :16]
    q = jnp.arange(4)
    w1s = w1p[:, 0, :8]
    sel1 = (30 * (q // 2)[:, None, None] + 2 * jnp.arange(14)[None, :, None]
            + (q % 2)[:, None, None])
    e1 = (jnp.arange(60)[None, None, :] == sel1).astype(jnp.float32)
    w1b = jnp.einsum('tc,qbr->qcbtr', w1s, e1).reshape(448, 540)
    w1b = jnp.pad(w1b, ((0, 0), (0, 4))).astype(bf16)
    w2k = jnp.transpose(w2p[:, :, :16], (1, 0, 2)).reshape(72, 16)
    sel2 = (16 * (q // 2)[:, None, None] + 2 * jnp.arange(7)[None, :, None]
            + (q % 2)[:, None, None])
    e2 = (jnp.arange(32)[None, None, :] == sel2).astype(jnp.float32)
    w2b = jnp.einsum('kc,qbr->qcbkr', w2k, e2).reshape(448, 2304).astype(bf16)
    wfc_t = jnp.transpose(wfc_p.reshape(49, 16, 10), (2, 1, 0)).reshape(10, 784)
    wfc16 = jnp.pad(wfc_t, ((0, 6), (0, 0))).astype(bf16)
    bfc16 = jnp.pad(bfc_p, ((0, 0), (0, 6)))
    bfcv = jnp.broadcast_to(bfc16.reshape(16, 1), (16, _N))

    out = pl.pallas_call(
        _cnn_kernel,
        out_shape=jax.ShapeDtypeStruct((G, 16, _N), jnp.float32),
        grid=(G,),
        in_specs=[
            pl.BlockSpec((784, _N), lambda g: (0, g)),
            pl.BlockSpec((448, 544), lambda g: (0, 0)),
            pl.BlockSpec(memory_space=pltpu.SMEM),
            pl.BlockSpec((448, 2304), lambda g: (0, 0)),
            pl.BlockSpec(memory_space=pltpu.SMEM),
            pl.BlockSpec((16, 784), lambda g: (0, 0)),
            pl.BlockSpec((16, _N), lambda g: (0, 0)),
        ],
        out_specs=pl.BlockSpec((None, 16, _N), lambda g: (g, 0, 0)),
        scratch_shapes=[
            pltpu.VMEM((904, _N), jnp.float32),
            pltpu.VMEM((2112, _N), jnp.float32),
            pltpu.VMEM((2, 544, _N), bf16),
            pltpu.VMEM((2, 448, _N), jnp.float32),
            pltpu.VMEM((2, 2304, _N), bf16),
            pltpu.VMEM((2, 448, _N), jnp.float32),
            pltpu.VMEM((784, _N), bf16),
        ],
        compiler_params=pltpu.CompilerParams(
            dimension_semantics=("arbitrary",)),
    )(xT, w1b, b1s, w2b, b2s, wfc16, bfcv)

    return jnp.transpose(out, (0, 2, 1)).reshape(B, 16)[:, :10]


_forward_jit = jax.jit(_forward)


def kernel(x, w1p, b1p, w2p, b2p, wfc_p, bfc_p):
    return _forward_jit(x, w1p, b1p, w2p, b2p, wfc_p, bfc_p)
